# baseline (speedup 1.0000x reference)
import functools
import jax
import jax.numpy as jnp
from jax import lax
from jax.experimental import pallas as pl
from jax.experimental.pallas import tpu as pltpu
from jax.experimental.pallas import tpu_sc as plsc

DIM = 64
SCALE = 8.0
LANES = 16

NC = 2
NS = 16
NW = NC * NS

CHUNK = 128


def _emb_body(nchunks, ids_hbm, table_hbm, out_hbm, idx_v, rows_v, sem):
    c = lax.axis_index("c")
    s = lax.axis_index("s")
    wid = s * NC + c
    b_per_w = nchunks * CHUNK
    base = wid * b_per_w

    pltpu.sync_copy(ids_hbm.at[wid], idx_v)

    def chunk_body(g, carry):
        pltpu.async_copy(table_hbm.at[idx_v.at[g]], rows_v, sem).wait()

        def row_body(r, rcarry):
            for cc in range(DIM // LANES):
                sl = pl.ds(cc * LANES, LANES)
                rows_v[r, sl] = rows_v[r, sl] * SCALE
            return rcarry

        lax.fori_loop(0, CHUNK, row_body, 0, unroll=4)
        pltpu.sync_copy(rows_v, out_hbm.at[pl.ds(base + g * CHUNK, CHUNK)])
        return carry

    lax.fori_loop(0, nchunks, chunk_body, 0)


@functools.partial(jax.jit, static_argnames=("batch", "seq"))
def _embed(ids3d, table, batch, seq):
    n_ids = batch * seq
    nchunks = n_ids // (NW * CHUNK)
    mesh = plsc.VectorSubcoreMesh(
        core_axis_name="c", subcore_axis_name="s", num_cores=NC,
        num_subcores=NS)
    out = pl.kernel(
        functools.partial(_emb_body, nchunks),
        out_type=jax.ShapeDtypeStruct((n_ids, DIM), jnp.float32),
        mesh=mesh,
        scratch_types=[
            pltpu.VMEM((nchunks, CHUNK), jnp.int32),
            pltpu.VMEM((CHUNK, DIM), jnp.float32),
            pltpu.SemaphoreType.DMA,
        ],
        compiler_params=pltpu.CompilerParams(use_tc_tiling_on_sc=False),
    )(ids3d, table)
    return out.reshape(batch, seq, DIM)


def kernel(input_ids, table):
    batch, seq = input_ids.shape
    n_ids = batch * seq
    ids3d = input_ids.reshape(NW, n_ids // (NW * CHUNK), CHUNK).astype(jnp.int32)
    return _embed(ids3d, table, batch, seq)


# speedup vs baseline: 1.0524x; 1.0524x over previous
import functools
import jax
import jax.numpy as jnp
from jax import lax
from jax.experimental import pallas as pl
from jax.experimental.pallas import tpu as pltpu
from jax.experimental.pallas import tpu_sc as plsc

DIM = 64
SCALE = 8.0
LANES = 16

NC = 2
NS = 16
NW = NC * NS

CHUNK = 128
NBUF = 4


def _emb_body(nchunks, ids_hbm, table_hbm, out_hbm, idx_v, in_v, out_v,
              gsems, ssems):
    c = lax.axis_index("c")
    s = lax.axis_index("s")
    wid = s * NC + c
    b_per_w = nchunks * CHUNK
    base = wid * b_per_w
    ngroups = nchunks // NBUF

    pltpu.sync_copy(ids_hbm.at[wid], idx_v)

    for b in range(NBUF):
        pltpu.async_copy(table_hbm.at[idx_v.at[b]], in_v.at[b], gsems[b])

    def group_body(go, carry):
        for b in range(NBUF):
            g = go * NBUF + b
            pltpu.make_async_copy(table_hbm.at[idx_v.at[b]], in_v.at[b],
                                  gsems[b]).wait()

            @pl.when(go > 0)
            def _():
                pltpu.make_async_copy(out_v.at[b], out_hbm.at[pl.ds(0, CHUNK)],
                                      ssems[b]).wait()

            def row_body(r, rc):
                for cc in range(DIM // LANES):
                    sl = pl.ds(cc * LANES, LANES)
                    out_v[b, r, sl] = in_v[b, r, sl] * SCALE
                return rc

            lax.fori_loop(0, CHUNK, row_body, 0, unroll=8)

            pltpu.async_copy(out_v.at[b],
                             out_hbm.at[pl.ds(base + g * CHUNK, CHUNK)],
                             ssems[b])

            @pl.when(go < ngroups - 1)
            def _():
                pltpu.async_copy(table_hbm.at[idx_v.at[g + NBUF]], in_v.at[b],
                                 gsems[b])

        return carry

    lax.fori_loop(0, ngroups, group_body, 0)

    for b in range(NBUF):
        pltpu.make_async_copy(out_v.at[b], out_hbm.at[pl.ds(0, CHUNK)],
                              ssems[b]).wait()


@functools.partial(jax.jit, static_argnames=("batch", "seq"))
def _embed(ids3d, table, batch, seq):
    n_ids = batch * seq
    nchunks = n_ids // (NW * CHUNK)
    mesh = plsc.VectorSubcoreMesh(
        core_axis_name="c", subcore_axis_name="s", num_cores=NC,
        num_subcores=NS)
    out = pl.kernel(
        functools.partial(_emb_body, nchunks),
        out_type=jax.ShapeDtypeStruct((n_ids, DIM), jnp.float32),
        mesh=mesh,
        scratch_types=[
            pltpu.VMEM((nchunks, CHUNK), jnp.int32),
            pltpu.VMEM((NBUF, CHUNK, DIM), jnp.float32),
            pltpu.VMEM((NBUF, CHUNK, DIM), jnp.float32),
            [pltpu.SemaphoreType.DMA] * NBUF,
            [pltpu.SemaphoreType.DMA] * NBUF,
        ],
        compiler_params=pltpu.CompilerParams(use_tc_tiling_on_sc=False),
    )(ids3d, table)
    return out.reshape(batch, seq, DIM)


def kernel(input_ids, table):
    batch, seq = input_ids.shape
    n_ids = batch * seq
    ids3d = input_ids.reshape(NW, n_ids // (NW * CHUNK), CHUNK).astype(jnp.int32)
    return _embed(ids3d, table, batch, seq)
